# Initial kernel scaffold; baseline (speedup 1.0000x reference)
import jax
import jax.numpy as jnp
from jax.experimental import pallas as pl


def kernel(x, edge_index, W_self0, W_neigh0, b0, W_self1, W_neigh1, b1, W_self2, W_neigh2, b2):
    raise NotImplementedError("write your pallas kernel here")


# speedup vs baseline: 3.6027x; 3.6027x over previous
import functools

import jax
import jax.numpy as jnp
from jax import lax
from jax.experimental import pallas as pl
from jax.experimental.pallas import tpu as pltpu
from jax.experimental.pallas import tpu_sc as plsc

N_NODES = 10000
N_EDGES = 160000
DIM = 256
HALF = 128
N_TILES = 16
EDGES_PER_TILE = N_EDGES // N_TILES
CHUNK = 80
N_CHUNKS = EDGES_PER_TILE // CHUNK
WB_TILES = 10
WB_ROWS = N_NODES // WB_TILES
ZROWS = 200
ZDEG = 1008


def _make_sc_agg(want_deg: bool):
    out_type = [
        jax.ShapeDtypeStruct((N_NODES, HALF), jnp.float32),
        jax.ShapeDtypeStruct((N_NODES, HALF), jnp.float32),
    ]
    if want_deg:
        out_type.append(jax.ShapeDtypeStruct((N_NODES,), jnp.float32))
    scratch = [
        pltpu.VMEM((CHUNK,), jnp.int32),
        pltpu.VMEM((CHUNK,), jnp.int32),
        pltpu.VMEM((CHUNK, HALF), jnp.float32),
        pltpu.VMEM((ZROWS, HALF), jnp.float32),
        pltpu.VMEM_SHARED((N_NODES, HALF), jnp.float32),
        pltpu.SemaphoreType.DMA,
    ]
    if want_deg:
        scratch += [
            pltpu.VMEM((CHUNK,), jnp.float32),
            pltpu.VMEM((ZDEG,), jnp.float32),
            pltpu.VMEM_SHARED((N_NODES,), jnp.float32),
        ]
    mesh = plsc.VectorSubcoreMesh(core_axis_name="c", subcore_axis_name="s")

    @functools.partial(pl.kernel, mesh=mesh, out_type=out_type,
                       scratch_types=scratch)
    def sc_agg(hL, hR, src_hbm, dst_hbm, *refs):
        if want_deg:
            (aggL, aggR, deg_out, idx_v, dst_v, rows_v, zbuf, acc, sem,
             ones_v, zdeg, dacc) = refs
        else:
            (aggL, aggR, idx_v, dst_v, rows_v, zbuf, acc, sem) = refs
        c = lax.axis_index("c")
        s = lax.axis_index("s")
        row0 = s * WB_ROWS

        @pl.when(s < WB_TILES)
        def _():
            def zf(i, carry):
                zbuf[i // 8, pl.ds((i % 8) * 16, 16)] = jnp.zeros((16,), jnp.float32)
                return carry
            lax.fori_loop(0, ZROWS * 8, zf, 0)
            for q in range(WB_ROWS // ZROWS):
                pltpu.sync_copy(zbuf, acc.at[pl.ds(row0 + q * ZROWS, ZROWS)])
        if want_deg:
            @pl.when(jnp.logical_and(c == 0, s < WB_TILES))
            def _():
                def df(i, carry):
                    zdeg[pl.ds(i * 16, 16)] = jnp.zeros((16,), jnp.float32)
                    return carry
                lax.fori_loop(0, ZDEG // 16, df, 0)
                pltpu.sync_copy(zdeg.at[pl.ds(0, WB_ROWS)],
                                dacc.at[pl.ds(row0, WB_ROWS)])

            @pl.when(c == 0)
            def _():
                def of(i, carry):
                    ones_v[pl.ds(i * 16, 16)] = jnp.ones((16,), jnp.float32)
                    return carry
                lax.fori_loop(0, CHUNK // 16, of, 0)
        plsc.subcore_barrier()

        ebase = s * EDGES_PER_TILE

        def chunk_body(k, carry):
            off = ebase + k * CHUNK
            pltpu.sync_copy(src_hbm.at[pl.ds(off, CHUNK)], idx_v)
            pltpu.sync_copy(dst_hbm.at[pl.ds(off, CHUNK)], dst_v)

            @pl.when(c == 0)
            def _():
                pltpu.async_copy(hL.at[idx_v], rows_v, sem).wait()

            @pl.when(c == 1)
            def _():
                pltpu.async_copy(hR.at[idx_v], rows_v, sem).wait()

            pltpu.sync_copy(rows_v, acc.at[dst_v], add=True)
            if want_deg:
                @pl.when(c == 0)
                def _():
                    pltpu.sync_copy(ones_v, dacc.at[dst_v], add=True)
            return carry
        lax.fori_loop(0, N_CHUNKS, chunk_body, 0)
        plsc.subcore_barrier()

        rows = pl.ds(row0, WB_ROWS)

        @pl.when(jnp.logical_and(c == 0, s < WB_TILES))
        def _():
            pltpu.sync_copy(acc.at[rows], aggL.at[rows])
            if want_deg:
                pltpu.sync_copy(dacc.at[rows], zdeg.at[pl.ds(0, WB_ROWS)])
                pltpu.sync_copy(zdeg.at[pl.ds(0, WB_ROWS)], deg_out.at[rows])

        @pl.when(jnp.logical_and(c == 1, s < WB_TILES))
        def _():
            pltpu.sync_copy(acc.at[rows], aggR.at[rows])

    return sc_agg


def _make_tc_layer(relu: bool, split_out: bool, rows_blk: int = 1000):
    grid = (N_NODES // rows_blk,)

    def body(hL, hR, aL, aR, dg, wsl, wsr, wnl, wnr, b, *o):
        inv = 1.0 / jnp.maximum(dg[...], 1.0)
        acc = jnp.dot(hL[...], wsl[...], preferred_element_type=jnp.float32)
        acc = acc + jnp.dot(hR[...], wsr[...], preferred_element_type=jnp.float32)
        acc = acc + jnp.dot(aL[...] * inv, wnl[...], preferred_element_type=jnp.float32)
        acc = acc + jnp.dot(aR[...] * inv, wnr[...], preferred_element_type=jnp.float32)
        acc = acc + b[...]
        if relu:
            acc = jnp.maximum(acc, 0.0)
        if split_out:
            o[0][...] = acc[:, :HALF]
            o[1][...] = acc[:, HALF:]
        else:
            o[0][...] = acc

    half_spec = pl.BlockSpec((rows_blk, HALF), lambda i: (i, 0))
    in_specs = [
        half_spec, half_spec, half_spec, half_spec,
        pl.BlockSpec((rows_blk, 1), lambda i: (i, 0)),
        pl.BlockSpec((HALF, DIM), lambda i: (0, 0)),
        pl.BlockSpec((HALF, DIM), lambda i: (0, 0)),
        pl.BlockSpec((HALF, DIM), lambda i: (0, 0)),
        pl.BlockSpec((HALF, DIM), lambda i: (0, 0)),
        pl.BlockSpec((1, DIM), lambda i: (0, 0)),
    ]
    if split_out:
        out_shape = [jax.ShapeDtypeStruct((N_NODES, HALF), jnp.float32)] * 2
        out_specs = [half_spec, half_spec]
    else:
        out_shape = [jax.ShapeDtypeStruct((N_NODES, DIM), jnp.float32)]
        out_specs = [pl.BlockSpec((rows_blk, DIM), lambda i: (i, 0))]
    return pl.pallas_call(body, grid=grid, in_specs=in_specs,
                          out_specs=out_specs, out_shape=out_shape)


_sc_agg_deg = _make_sc_agg(want_deg=True)
_sc_agg = _make_sc_agg(want_deg=False)
_tc_hidden = _make_tc_layer(relu=True, split_out=True)
_tc_final = _make_tc_layer(relu=False, split_out=False)


def _split(w):
    return w[:HALF], w[HALF:]


def kernel(x, edge_index, W_self0, W_neigh0, b0, W_self1, W_neigh1, b1,
           W_self2, W_neigh2, b2):
    src = edge_index[0].astype(jnp.int32)
    dst = edge_index[1].astype(jnp.int32)
    xL, xR = x[:, :HALF], x[:, HALF:]

    a1L, a1R, deg1d = _sc_agg_deg(xL, xR, src, dst)
    deg = deg1d.reshape(N_NODES, 1)
    h1L, h1R = _tc_hidden(xL, xR, a1L, a1R, deg,
                          *_split(W_self0), *_split(W_neigh0),
                          b0.reshape(1, DIM))
    a2L, a2R = _sc_agg(h1L, h1R, src, dst)
    h2L, h2R = _tc_hidden(h1L, h1R, a2L, a2R, deg,
                          *_split(W_self1), *_split(W_neigh1),
                          b1.reshape(1, DIM))
    a3L, a3R = _sc_agg(h2L, h2R, src, dst)
    (out,) = _tc_final(h2L, h2R, a3L, a3R, deg,
                       *_split(W_self2), *_split(W_neigh2),
                       b2.reshape(1, DIM))
    return out


# speedup vs baseline: 4.1575x; 1.1540x over previous
import functools

import jax
import jax.numpy as jnp
from jax import lax
from jax.experimental import pallas as pl
from jax.experimental.pallas import tpu as pltpu
from jax.experimental.pallas import tpu_sc as plsc

N_NODES = 10000
N_EDGES = 160000
DIM = 256
HALF = 128
N_TILES = 16
EDGES_PER_TILE = N_EDGES // N_TILES
CHUNK = 40
N_CHUNKS = EDGES_PER_TILE // CHUNK
WB_TILES = 10
WB_ROWS = N_NODES // WB_TILES
ZROWS = 40
ZDEG = 1008
NSLOT = 5


def _make_sc_agg(want_deg: bool):
    out_type = [
        jax.ShapeDtypeStruct((N_NODES, HALF), jnp.float32),
        jax.ShapeDtypeStruct((N_NODES, HALF), jnp.float32),
    ]
    if want_deg:
        out_type.append(jax.ShapeDtypeStruct((N_NODES,), jnp.float32))
    scratch = (
        [pltpu.VMEM((CHUNK,), jnp.int32) for _ in range(NSLOT)]
        + [pltpu.VMEM((CHUNK,), jnp.int32) for _ in range(NSLOT)]
        + [pltpu.VMEM((CHUNK, HALF), jnp.float32) for _ in range(NSLOT)]
        + [
            pltpu.VMEM((ZROWS, HALF), jnp.float32),
            pltpu.VMEM_SHARED((N_NODES, HALF), jnp.float32),
        ]
        + [pltpu.SemaphoreType.DMA for _ in range(2 * NSLOT)]
    )
    if want_deg:
        scratch += [
            pltpu.VMEM((48,), jnp.float32),
            pltpu.VMEM((ZDEG,), jnp.float32),
            pltpu.VMEM_SHARED((N_NODES,), jnp.float32),
        ] + [pltpu.SemaphoreType.DMA for _ in range(NSLOT)]
    mesh = plsc.VectorSubcoreMesh(core_axis_name="c", subcore_axis_name="s")

    @functools.partial(pl.kernel, mesh=mesh, out_type=out_type,
                       scratch_types=scratch)
    def sc_agg(hL, hR, src_hbm, dst_hbm, *refs):
        if want_deg:
            aggL, aggR, deg_out = refs[:3]
            refs = refs[3:]
        else:
            aggL, aggR = refs[:2]
            refs = refs[2:]
        idx_l = refs[0:NSLOT]
        dst_l = refs[NSLOT:2 * NSLOT]
        rows_l = refs[2 * NSLOT:3 * NSLOT]
        zbuf, acc = refs[3 * NSLOT:3 * NSLOT + 2]
        gsem = refs[3 * NSLOT + 2:4 * NSLOT + 2]
        ssem = refs[4 * NSLOT + 2:5 * NSLOT + 2]
        if want_deg:
            ones_v, zdeg, dacc = refs[5 * NSLOT + 2:5 * NSLOT + 5]
            dsem = refs[5 * NSLOT + 5:6 * NSLOT + 5]
        c = lax.axis_index("c")
        s = lax.axis_index("s")
        row0 = s * WB_ROWS

        @pl.when(s < WB_TILES)
        def _():
            def zf(i, carry):
                zbuf[i // 8, pl.ds((i % 8) * 16, 16)] = jnp.zeros((16,), jnp.float32)
                return carry
            lax.fori_loop(0, ZROWS * 8, zf, 0)
            for q in range(WB_ROWS // ZROWS):
                pltpu.sync_copy(zbuf, acc.at[pl.ds(row0 + q * ZROWS, ZROWS)])
        if want_deg:
            @pl.when(jnp.logical_and(c == 0, s < WB_TILES))
            def _():
                def df(i, carry):
                    zdeg[pl.ds(i * 16, 16)] = jnp.zeros((16,), jnp.float32)
                    return carry
                lax.fori_loop(0, ZDEG // 16, df, 0)
                pltpu.sync_copy(zdeg.at[pl.ds(0, WB_ROWS)],
                                dacc.at[pl.ds(row0, WB_ROWS)])

            @pl.when(c == 0)
            def _():
                def of(i, carry):
                    ones_v[pl.ds(i * 16, 16)] = jnp.ones((16,), jnp.float32)
                    return carry
                lax.fori_loop(0, 48 // 16, of, 0)
        plsc.subcore_barrier()

        ebase = s * EDGES_PER_TILE

        def _drain_slot(j):
            pltpu.make_async_copy(rows_l[j], acc.at[dst_l[j]], ssem[j]).wait()
            if want_deg:
                @pl.when(c == 0)
                def _():
                    pltpu.make_async_copy(ones_v.at[pl.ds(0, CHUNK)],
                                          dacc.at[dst_l[j]], dsem[j]).wait()

        def group_body(g, carry):
            base = ebase + g * (NSLOT * CHUNK)
            for j in range(NSLOT):
                @pl.when(g > 0)
                def _(j=j):
                    _drain_slot(j)
                off = base + j * CHUNK
                pltpu.sync_copy(src_hbm.at[pl.ds(off, CHUNK)], idx_l[j])
                pltpu.sync_copy(dst_hbm.at[pl.ds(off, CHUNK)], dst_l[j])

                @pl.when(c == 0)
                def _(j=j):
                    pltpu.async_copy(hL.at[idx_l[j]], rows_l[j], gsem[j])

                @pl.when(c == 1)
                def _(j=j):
                    pltpu.async_copy(hR.at[idx_l[j]], rows_l[j], gsem[j])
            for j in range(NSLOT):
                pltpu.make_async_copy(hL.at[idx_l[j]], rows_l[j],
                                      gsem[j]).wait()
                pltpu.async_copy(rows_l[j], acc.at[dst_l[j]], ssem[j],
                                 add=True)
                if want_deg:
                    @pl.when(c == 0)
                    def _(j=j):
                        pltpu.async_copy(ones_v.at[pl.ds(0, CHUNK)],
                                         dacc.at[dst_l[j]], dsem[j],
                                         add=True)
            return carry
        lax.fori_loop(0, N_CHUNKS // NSLOT, group_body, 0)
        for j in range(NSLOT):
            _drain_slot(j)
        plsc.subcore_barrier()

        rows = pl.ds(row0, WB_ROWS)

        @pl.when(jnp.logical_and(c == 0, s < WB_TILES))
        def _():
            pltpu.sync_copy(acc.at[rows], aggL.at[rows])
            if want_deg:
                pltpu.sync_copy(dacc.at[rows], zdeg.at[pl.ds(0, WB_ROWS)])
                pltpu.sync_copy(zdeg.at[pl.ds(0, WB_ROWS)], deg_out.at[rows])

        @pl.when(jnp.logical_and(c == 1, s < WB_TILES))
        def _():
            pltpu.sync_copy(acc.at[rows], aggR.at[rows])

    return sc_agg


def _make_tc_layer(relu: bool, split_out: bool, rows_blk: int = 1000):
    grid = (N_NODES // rows_blk,)

    def body(hL, hR, aL, aR, dg, wsl, wsr, wnl, wnr, b, *o):
        inv = 1.0 / jnp.maximum(dg[...], 1.0)
        acc = jnp.dot(hL[...], wsl[...], preferred_element_type=jnp.float32)
        acc = acc + jnp.dot(hR[...], wsr[...], preferred_element_type=jnp.float32)
        acc = acc + jnp.dot(aL[...] * inv, wnl[...], preferred_element_type=jnp.float32)
        acc = acc + jnp.dot(aR[...] * inv, wnr[...], preferred_element_type=jnp.float32)
        acc = acc + b[...]
        if relu:
            acc = jnp.maximum(acc, 0.0)
        if split_out:
            o[0][...] = acc[:, :HALF]
            o[1][...] = acc[:, HALF:]
        else:
            o[0][...] = acc

    half_spec = pl.BlockSpec((rows_blk, HALF), lambda i: (i, 0))
    in_specs = [
        half_spec, half_spec, half_spec, half_spec,
        pl.BlockSpec((rows_blk, 1), lambda i: (i, 0)),
        pl.BlockSpec((HALF, DIM), lambda i: (0, 0)),
        pl.BlockSpec((HALF, DIM), lambda i: (0, 0)),
        pl.BlockSpec((HALF, DIM), lambda i: (0, 0)),
        pl.BlockSpec((HALF, DIM), lambda i: (0, 0)),
        pl.BlockSpec((1, DIM), lambda i: (0, 0)),
    ]
    if split_out:
        out_shape = [jax.ShapeDtypeStruct((N_NODES, HALF), jnp.float32)] * 2
        out_specs = [half_spec, half_spec]
    else:
        out_shape = [jax.ShapeDtypeStruct((N_NODES, DIM), jnp.float32)]
        out_specs = [pl.BlockSpec((rows_blk, DIM), lambda i: (i, 0))]
    return pl.pallas_call(body, grid=grid, in_specs=in_specs,
                          out_specs=out_specs, out_shape=out_shape)


_sc_agg_deg = _make_sc_agg(want_deg=True)
_sc_agg = _make_sc_agg(want_deg=False)
_tc_hidden = _make_tc_layer(relu=True, split_out=True)
_tc_final = _make_tc_layer(relu=False, split_out=False)


def _split(w):
    return w[:HALF], w[HALF:]


def kernel(x, edge_index, W_self0, W_neigh0, b0, W_self1, W_neigh1, b1,
           W_self2, W_neigh2, b2):
    src = edge_index[0].astype(jnp.int32)
    dst = edge_index[1].astype(jnp.int32)
    xL, xR = x[:, :HALF], x[:, HALF:]

    a1L, a1R, deg1d = _sc_agg_deg(xL, xR, src, dst)
    deg = deg1d.reshape(N_NODES, 1)
    h1L, h1R = _tc_hidden(xL, xR, a1L, a1R, deg,
                          *_split(W_self0), *_split(W_neigh0),
                          b0.reshape(1, DIM))
    a2L, a2R = _sc_agg(h1L, h1R, src, dst)
    h2L, h2R = _tc_hidden(h1L, h1R, a2L, a2R, deg,
                          *_split(W_self1), *_split(W_neigh1),
                          b1.reshape(1, DIM))
    a3L, a3R = _sc_agg(h2L, h2R, src, dst)
    (out,) = _tc_final(h2L, h2R, a3L, a3R, deg,
                       *_split(W_self2), *_split(W_neigh2),
                       b2.reshape(1, DIM))
    return out
